# baseline (speedup 1.0000x reference)
import jax
import jax.numpy as jnp
from jax import lax
from jax.experimental import pallas as pl
from jax.experimental.pallas import tpu as pltpu
from jax.experimental.pallas import tpu_sc as plsc

D_MODEL = 1024
MEMORY_SLOTS = 100000
HASH_BASE_MOD = 1315423911 % MEMORY_SLOTS
EPS = 1e-6

LANES = 16
ROWS_PER_WORKER = 512
CHUNK = 16
NUM_CHUNKS = ROWS_PER_WORKER // CHUNK
VECS_PER_ROW = D_MODEL // LANES
PAD = 8

i32 = jnp.int32


def _hash16(ids_ref, base):
    a = ids_ref[pl.ds(base, LANES)].astype(jnp.uint32)
    b = ids_ref[pl.ds(base + 1, LANES)].astype(jnp.uint32)
    c = ids_ref[pl.ds(base + 2, LANES)].astype(jnp.uint32)
    m = jnp.uint32(MEMORY_SLOTS)
    h = jnp.uint32(HASH_BASE_MOD)
    s = (a * h + b) % m
    s = (s * h + c) % m
    return s.astype(i32)


_GATHER_1D = lax.GatherDimensionNumbers(
    offset_dims=(), collapsed_slice_dims=(0,), start_index_map=(0,))


def _take16(v, idx):
    return lax.gather(v, idx[:, None], _GATHER_1D, slice_sizes=(1,),
                      mode=lax.GatherScatterMode.PROMISE_IN_BOUNDS)


def _sum_lanes(v):
    lanes = lax.iota(i32, 16)
    for s in (8, 4, 2, 1):
        v = v + _take16(v, lanes ^ s)
    return v


def _rsqrt16(x):
    i = plsc.bitcast(x, i32)
    y = plsc.bitcast(i32(0x5F3759DF) - (i >> 1), jnp.float32)
    half_x = x * 0.5
    for _ in range(3):
        y = y * (1.5 - half_x * y * y)
    return y


def _scales_chunk(mem_b, scales_v, g):
    def scale_quad(r4, _):
        r0 = r4 * i32(4)
        rows = [r0, r0 + i32(1), r0 + i32(2), r0 + i32(3)]
        accs = [[jnp.zeros((LANES,), jnp.float32) for _ in range(2)]
                for _ in rows]
        for k in range(0, VECS_PER_ROW, 2):
            for ri, r in enumerate(rows):
                for j in range(2):
                    x = mem_b[r, pl.ds((k + j) * LANES, LANES)]
                    accs[ri][j] = accs[ri][j] + x * x
        for ri, r in enumerate(rows):
            var = _sum_lanes(accs[ri][0] + accs[ri][1]) * (1.0 / D_MODEL)
            scales_v[r, pl.ds(0, LANES)] = _rsqrt16(var + EPS) * g
        return 0

    lax.fori_loop(i32(0), i32(CHUNK // 4), scale_quad, 0)


def _blend_chunk(mem_b, hid_b, w_v, scales_v):
    for half in range(2):
        rows = range(half * (CHUNK // 2), (half + 1) * (CHUNK // 2))
        scs = {r: scales_v[r, pl.ds(0, LANES)] for r in rows}

        def kb_body(kb, carry):
            base = kb * i32(8 * LANES)
            for j in range(8):
                sl = pl.ds(base + i32(j * LANES), LANES)
                wk = w_v[sl]
                for r in rows:
                    plsc.addupdate(hid_b.at[r, sl],
                                   mem_b[r, sl] * (wk * carry[r]))
            return carry

        lax.fori_loop(i32(0), i32(VECS_PER_ROW // 8), kb_body, scs)


def _sc_body(ids_hbm, hid_hbm, mem_hbm, w_hbm, gate_hbm, out_hbm,
             ids_v, slots_v, mem0, mem1, hid0, hid1, hid2, hid3,
             w_v, gate_v, scales_v,
             sm0, sm1, sh0, sh1, sh2, sh3, so0, so1, so2, so3):
    wid = lax.axis_index("c") * i32(16) + lax.axis_index("s")
    row0 = wid * i32(ROWS_PER_WORKER)
    b = wid // i32(8)
    t0 = (wid % i32(8)) * i32(ROWS_PER_WORKER)

    ids_base = b * i32(4096 + PAD) + t0
    pltpu.sync_copy(ids_hbm.at[pl.ds(ids_base, ROWS_PER_WORKER + PAD)], ids_v)
    pltpu.sync_copy(w_hbm, w_v)
    pltpu.sync_copy(gate_hbm, gate_v)

    gate16 = gate_v[...]
    g = 1.0 / (1.0 + jnp.exp(-gate16))

    for blk in range(ROWS_PER_WORKER // LANES):
        s = _hash16(ids_v, blk * LANES + PAD - 2)
        slots_v[blk, pl.ds(0, LANES)] = s

    mems = [mem0, mem1]
    hids = [hid0, hid1, hid2, hid3]
    sms = [sm0, sm1]
    shs = [sh0, sh1, sh2, sh3]
    sos = [so0, so1, so2, so3]

    def issue_mem(c, mb, sem_m):
        pltpu.async_copy(mem_hbm.at[slots_v.at[c]], mb, sem_m)

    def wait_mem(c, mb, sem_m):
        pltpu.make_async_copy(mem_hbm.at[slots_v.at[c]], mb, sem_m).wait()

    def issue_hid(c, hb, sem_h):
        rbase = row0 + c * i32(CHUNK)
        pltpu.async_copy(hid_hbm.at[pl.ds(rbase, CHUNK)], hb, sem_h)

    def wait_hid(c, hb, sem_h):
        rbase = row0 + c * i32(CHUNK)
        pltpu.make_async_copy(
            hid_hbm.at[pl.ds(rbase, CHUNK)], hb, sem_h).wait()

    def put_out(c, hb, sem_o):
        rbase = row0 + c * i32(CHUNK)
        pltpu.async_copy(hb, out_hbm.at[pl.ds(rbase, CHUNK)], sem_o)

    def wait_out(c, hb, sem_o):
        rbase = row0 + c * i32(CHUNK)
        pltpu.make_async_copy(
            hb, out_hbm.at[pl.ds(rbase, CHUNK)], sem_o).wait()

    issue_mem(i32(0), mem0, sm0)
    issue_mem(i32(1), mem1, sm1)
    issue_hid(i32(0), hid0, sh0)
    issue_hid(i32(1), hid1, sh1)

    def quad_body(c4, _):
        for p in range(4):
            c = c4 * i32(4) + i32(p)
            mb, sem_m = mems[p % 2], sms[p % 2]
            hb, sem_h = hids[p], shs[p]
            wait_mem(c, mb, sem_m)
            _scales_chunk(mb, scales_v, g)
            wait_hid(c, hb, sem_h)
            _blend_chunk(mb, hb, w_v, scales_v)
            put_out(c, hb, sos[p])

            pn = (p + 2) % 4

            if p < 2:
                issue_mem(c + i32(2), mems[pn % 2], sms[pn % 2])

                @pl.when(c4 > i32(0))
                def _():
                    wait_out(c - i32(2), hids[pn], sos[pn])

                issue_hid(c + i32(2), hids[pn], shs[pn])
            else:
                @pl.when(c4 < i32(NUM_CHUNKS // 4 - 1))
                def _():
                    issue_mem(c + i32(2), mems[pn % 2], sms[pn % 2])

                wait_out(c - i32(2), hids[pn], sos[pn])

                @pl.when(c4 < i32(NUM_CHUNKS // 4 - 1))
                def _():
                    issue_hid(c + i32(2), hids[pn], shs[pn])
        return 0

    lax.fori_loop(i32(0), i32(NUM_CHUNKS // 4), quad_body, 0)
    wait_out(i32(NUM_CHUNKS - 2), hid2, so2)
    wait_out(i32(NUM_CHUNKS - 1), hid3, so3)


def kernel(input_ids, hidden, memory, norm_weight, gate):
    B, T = input_ids.shape
    N = B * T
    ids32 = input_ids.astype(i32)
    ids_pad = jnp.zeros((B, T + PAD), i32).at[:, PAD:].set(ids32)
    ids_pad = ids_pad.reshape(B * (T + PAD))
    hid2 = hidden.reshape(N, D_MODEL)
    gate16 = jnp.broadcast_to(gate.astype(jnp.float32), (LANES,))

    mesh = plsc.VectorSubcoreMesh(core_axis_name="c", subcore_axis_name="s")
    fn = pl.kernel(
        _sc_body,
        out_type=jax.ShapeDtypeStruct((N, D_MODEL), jnp.float32),
        mesh=mesh,
        compiler_params=pltpu.CompilerParams(needs_layout_passes=False),
        scratch_types=[
            pltpu.VMEM((ROWS_PER_WORKER + PAD,), i32),
            pltpu.VMEM((NUM_CHUNKS, CHUNK), i32),
            pltpu.VMEM((CHUNK, D_MODEL), jnp.float32),
            pltpu.VMEM((CHUNK, D_MODEL), jnp.float32),
            pltpu.VMEM((CHUNK, D_MODEL), jnp.float32),
            pltpu.VMEM((CHUNK, D_MODEL), jnp.float32),
            pltpu.VMEM((CHUNK, D_MODEL), jnp.float32),
            pltpu.VMEM((CHUNK, D_MODEL), jnp.float32),
            pltpu.VMEM((D_MODEL,), jnp.float32),
            pltpu.VMEM((LANES,), jnp.float32),
            pltpu.VMEM((CHUNK, LANES), jnp.float32),
        ] + [pltpu.SemaphoreType.DMA] * 10,
    )
    out2 = fn(ids_pad, hid2, memory, norm_weight.astype(jnp.float32), gate16)
    return out2.reshape(B, T, D_MODEL)


# speedup vs baseline: 1.0187x; 1.0187x over previous
import jax
import jax.numpy as jnp
from jax import lax
from jax.experimental import pallas as pl
from jax.experimental.pallas import tpu as pltpu
from jax.experimental.pallas import tpu_sc as plsc

D_MODEL = 1024
MEMORY_SLOTS = 100000
HASH_BASE_MOD = 1315423911 % MEMORY_SLOTS
EPS = 1e-6

LANES = 16
ROWS_PER_WORKER = 512
CHUNK = 16
NUM_CHUNKS = ROWS_PER_WORKER // CHUNK
VECS_PER_ROW = D_MODEL // LANES
PAD = 8

i32 = jnp.int32


def _hash16(ids_ref, base):
    a = ids_ref[pl.ds(base, LANES)].astype(jnp.uint32)
    b = ids_ref[pl.ds(base + 1, LANES)].astype(jnp.uint32)
    c = ids_ref[pl.ds(base + 2, LANES)].astype(jnp.uint32)
    m = jnp.uint32(MEMORY_SLOTS)
    h = jnp.uint32(HASH_BASE_MOD)
    s = (a * h + b) % m
    s = (s * h + c) % m
    return s.astype(i32)


_GATHER_1D = lax.GatherDimensionNumbers(
    offset_dims=(), collapsed_slice_dims=(0,), start_index_map=(0,))


def _take16(v, idx):
    return lax.gather(v, idx[:, None], _GATHER_1D, slice_sizes=(1,),
                      mode=lax.GatherScatterMode.PROMISE_IN_BOUNDS)


def _sum_lanes(v):
    lanes = lax.iota(i32, 16)
    for s in (8, 4, 2, 1):
        v = v + _take16(v, lanes ^ s)
    return v


def _rsqrt16(x):
    i = plsc.bitcast(x, i32)
    y = plsc.bitcast(i32(0x5F3759DF) - (i >> 1), jnp.float32)
    half_x = x * 0.5
    for _ in range(3):
        y = y * (1.5 - half_x * y * y)
    return y


def _scales_chunk(mem_b, scales_v, g):
    def scale_quad(r4, _):
        r0 = r4 * i32(4)
        rows = [r0, r0 + i32(1), r0 + i32(2), r0 + i32(3)]
        accs = [[jnp.zeros((LANES,), jnp.float32) for _ in range(2)]
                for _ in rows]
        for k in range(0, VECS_PER_ROW, 2):
            for ri, r in enumerate(rows):
                for j in range(2):
                    x = mem_b[r, pl.ds((k + j) * LANES, LANES)]
                    accs[ri][j] = accs[ri][j] + x * x
        for ri, r in enumerate(rows):
            var = _sum_lanes(accs[ri][0] + accs[ri][1]) * (1.0 / D_MODEL)
            scales_v[r, pl.ds(0, LANES)] = _rsqrt16(var + EPS) * g
        return 0

    lax.fori_loop(i32(0), i32(CHUNK // 4), scale_quad, 0)


def _blend_chunk(mem_b, hid_b, w_v, scales_v):
    for half in range(2):
        rows = range(half * (CHUNK // 2), (half + 1) * (CHUNK // 2))
        scs = {r: scales_v[r, pl.ds(0, LANES)] for r in rows}

        def kb_body(kb, carry):
            base = kb * i32(8 * LANES)
            for j0 in range(0, 8, 2):
                sls = [pl.ds(base + i32((j0 + j) * LANES), LANES)
                       for j in range(2)]
                wks = [w_v[sl] for sl in sls]
                zs = [[mem_b[r, sls[j]] * (wks[j] * carry[r]) for r in rows]
                      for j in range(2)]
                for j in range(2):
                    for ri, r in enumerate(rows):
                        plsc.addupdate(hid_b.at[r, sls[j]], zs[j][ri])
            return carry

        lax.fori_loop(i32(0), i32(VECS_PER_ROW // 8), kb_body, scs)


def _sc_body(ids_hbm, hid_hbm, mem_hbm, w_hbm, gate_hbm, out_hbm,
             ids_v, slots_v, mem0, mem1, hid0, hid1, hid2, hid3,
             w_v, gate_v, scales_v,
             sm0, sm1, sh0, sh1, sh2, sh3, so0, so1, so2, so3):
    wid = lax.axis_index("c") * i32(16) + lax.axis_index("s")
    row0 = wid * i32(ROWS_PER_WORKER)
    b = wid // i32(8)
    t0 = (wid % i32(8)) * i32(ROWS_PER_WORKER)

    ids_base = b * i32(4096 + PAD) + t0
    pltpu.sync_copy(ids_hbm.at[pl.ds(ids_base, ROWS_PER_WORKER + PAD)], ids_v)
    pltpu.sync_copy(w_hbm, w_v)
    pltpu.sync_copy(gate_hbm, gate_v)

    gate16 = gate_v[...]
    g = 1.0 / (1.0 + jnp.exp(-gate16))

    for blk in range(ROWS_PER_WORKER // LANES):
        s = _hash16(ids_v, blk * LANES + PAD - 2)
        slots_v[blk, pl.ds(0, LANES)] = s

    mems = [mem0, mem1]
    hids = [hid0, hid1, hid2, hid3]
    sms = [sm0, sm1]
    shs = [sh0, sh1, sh2, sh3]
    sos = [so0, so1, so2, so3]

    def issue_mem(c, mb, sem_m):
        pltpu.async_copy(mem_hbm.at[slots_v.at[c]], mb, sem_m)

    def wait_mem(c, mb, sem_m):
        pltpu.make_async_copy(mem_hbm.at[slots_v.at[c]], mb, sem_m).wait()

    def issue_hid(c, hb, sem_h):
        rbase = row0 + c * i32(CHUNK)
        pltpu.async_copy(hid_hbm.at[pl.ds(rbase, CHUNK)], hb, sem_h)

    def wait_hid(c, hb, sem_h):
        rbase = row0 + c * i32(CHUNK)
        pltpu.make_async_copy(
            hid_hbm.at[pl.ds(rbase, CHUNK)], hb, sem_h).wait()

    def put_out(c, hb, sem_o):
        rbase = row0 + c * i32(CHUNK)
        pltpu.async_copy(hb, out_hbm.at[pl.ds(rbase, CHUNK)], sem_o)

    def wait_out(c, hb, sem_o):
        rbase = row0 + c * i32(CHUNK)
        pltpu.make_async_copy(
            hb, out_hbm.at[pl.ds(rbase, CHUNK)], sem_o).wait()

    issue_mem(i32(0), mem0, sm0)
    issue_mem(i32(1), mem1, sm1)
    issue_hid(i32(0), hid0, sh0)
    issue_hid(i32(1), hid1, sh1)

    def quad_body(c4, _):
        for p in range(4):
            c = c4 * i32(4) + i32(p)
            mb, sem_m = mems[p % 2], sms[p % 2]
            hb, sem_h = hids[p], shs[p]
            wait_mem(c, mb, sem_m)
            _scales_chunk(mb, scales_v, g)
            wait_hid(c, hb, sem_h)
            _blend_chunk(mb, hb, w_v, scales_v)
            put_out(c, hb, sos[p])

            pn = (p + 2) % 4

            if p < 2:
                issue_mem(c + i32(2), mems[pn % 2], sms[pn % 2])

                @pl.when(c4 > i32(0))
                def _():
                    wait_out(c - i32(2), hids[pn], sos[pn])

                issue_hid(c + i32(2), hids[pn], shs[pn])
            else:
                @pl.when(c4 < i32(NUM_CHUNKS // 4 - 1))
                def _():
                    issue_mem(c + i32(2), mems[pn % 2], sms[pn % 2])

                wait_out(c - i32(2), hids[pn], sos[pn])

                @pl.when(c4 < i32(NUM_CHUNKS // 4 - 1))
                def _():
                    issue_hid(c + i32(2), hids[pn], shs[pn])
        return 0

    lax.fori_loop(i32(0), i32(NUM_CHUNKS // 4), quad_body, 0)
    wait_out(i32(NUM_CHUNKS - 2), hid2, so2)
    wait_out(i32(NUM_CHUNKS - 1), hid3, so3)


def kernel(input_ids, hidden, memory, norm_weight, gate):
    B, T = input_ids.shape
    N = B * T
    ids32 = input_ids.astype(i32)
    ids_pad = jnp.zeros((B, T + PAD), i32).at[:, PAD:].set(ids32)
    ids_pad = ids_pad.reshape(B * (T + PAD))
    hid2 = hidden.reshape(N, D_MODEL)
    gate16 = jnp.broadcast_to(gate.astype(jnp.float32), (LANES,))

    mesh = plsc.VectorSubcoreMesh(core_axis_name="c", subcore_axis_name="s")
    fn = pl.kernel(
        _sc_body,
        out_type=jax.ShapeDtypeStruct((N, D_MODEL), jnp.float32),
        mesh=mesh,
        compiler_params=pltpu.CompilerParams(needs_layout_passes=False),
        scratch_types=[
            pltpu.VMEM((ROWS_PER_WORKER + PAD,), i32),
            pltpu.VMEM((NUM_CHUNKS, CHUNK), i32),
            pltpu.VMEM((CHUNK, D_MODEL), jnp.float32),
            pltpu.VMEM((CHUNK, D_MODEL), jnp.float32),
            pltpu.VMEM((CHUNK, D_MODEL), jnp.float32),
            pltpu.VMEM((CHUNK, D_MODEL), jnp.float32),
            pltpu.VMEM((CHUNK, D_MODEL), jnp.float32),
            pltpu.VMEM((CHUNK, D_MODEL), jnp.float32),
            pltpu.VMEM((D_MODEL,), jnp.float32),
            pltpu.VMEM((LANES,), jnp.float32),
            pltpu.VMEM((CHUNK, LANES), jnp.float32),
        ] + [pltpu.SemaphoreType.DMA] * 10,
    )
    out2 = fn(ids_pad, hid2, memory, norm_weight.astype(jnp.float32), gate16)
    return out2.reshape(B, T, D_MODEL)
